# baseline (speedup 1.0000x reference)
import functools

import jax
import jax.numpy as jnp
from jax import lax
from jax.experimental import pallas as pl
from jax.experimental.pallas import tpu as pltpu
from jax.experimental.pallas import tpu_sc as plsc

N_BINS = 11
L = 16
NW = 32
CHUNK_R = 64
NEG = -3.0e38
SPLIT = 8192


def _stats(x, labels, lo, hi):
    m = jnp.max(x, axis=1, keepdims=True)
    s = jnp.sum(jnp.exp(x - m), axis=1, keepdims=True)
    conf = 1.0 / s
    col = jax.lax.broadcasted_iota(jnp.int32, x.shape, 1)
    xl = jnp.max(jnp.where(col == labels, x, NEG), axis=1, keepdims=True)
    acc = (xl == m).astype(jnp.float32)
    mask = ((conf > lo) & (conf <= hi)).astype(jnp.float32)
    return (jnp.sum(mask, axis=0, keepdims=True),
            jnp.sum(mask * acc, axis=0, keepdims=True),
            jnp.sum(mask * conf, axis=0, keepdims=True))


def _tc_partial_kernel(nb, xa_ref, xb_ref, labels_ref, bounds_ref,
                       cnt_ref, asum_ref, csum_ref, cnt_s, asum_s, csum_s):
    i = pl.program_id(0)

    @pl.when(i == 0)
    def _init():
        cnt_s[...] = jnp.zeros_like(cnt_s)
        asum_s[...] = jnp.zeros_like(asum_s)
        csum_s[...] = jnp.zeros_like(csum_s)

    lo = bounds_ref[0:1, 0:N_BINS]
    hi = bounds_ref[0:1, 1:N_BINS + 1]
    half = xa_ref.shape[0]
    labs = labels_ref[0]

    c1, a1, s1 = _stats(xa_ref[...], labs[:half], lo, hi)
    c2, a2, s2 = _stats(xb_ref[...], labs[half:], lo, hi)
    cnt_s[...] += c1 + c2
    asum_s[...] += a1 + a2
    csum_s[...] += s1 + s2

    @pl.when(i == nb - 1)
    def _fin():
        cnt_ref[...] = cnt_s[...]
        asum_ref[...] = asum_s[...]
        csum_ref[...] = csum_s[...]


def _tc_partial(logits, labels3, bounds, nb, block_r):
    n_cols = logits.shape[1]
    return pl.pallas_call(
        functools.partial(_tc_partial_kernel, nb),
        grid=(nb,),
        in_specs=[
            pl.BlockSpec((block_r, n_cols), lambda i: (2 * i, 0)),
            pl.BlockSpec((block_r, n_cols), lambda i: (2 * i + 1, 0)),
            pl.BlockSpec((1, 2 * block_r, 1), lambda i: (i, 0, 0)),
            pl.BlockSpec((1, N_BINS + 1), lambda i: (0, 0)),
        ],
        out_specs=[
            pl.BlockSpec((1, N_BINS), lambda i: (0, 0)),
            pl.BlockSpec((1, N_BINS), lambda i: (0, 0)),
            pl.BlockSpec((1, N_BINS), lambda i: (0, 0)),
        ],
        out_shape=[
            jax.ShapeDtypeStruct((1, N_BINS), jnp.float32),
            jax.ShapeDtypeStruct((1, N_BINS), jnp.float32),
            jax.ShapeDtypeStruct((1, N_BINS), jnp.float32),
        ],
        scratch_shapes=[
            pltpu.VMEM((1, N_BINS), jnp.float32),
            pltpu.VMEM((1, N_BINS), jnp.float32),
            pltpu.VMEM((1, N_BINS), jnp.float32),
        ],
    )(logits, logits, labels3, bounds)


def _perm(v, idx):
    return lax.gather(
        v, idx[:, None],
        lax.GatherDimensionNumbers(
            offset_dims=(), collapsed_slice_dims=(0,), start_index_map=(0,)),
        slice_sizes=(1,),
        mode=lax.GatherScatterMode.PROMISE_IN_BOUNDS)


def _butterfly(v, op):
    for sh in (8, 4, 2, 1):
        idx = (jnp.arange(16, dtype=jnp.int32) + sh) % 16
        v = op(v, _perm(v, idx))
    return v


def _make_sc_partial(n_rows_sc, row0_global, n_cols):
    rows_per_w = n_rows_sc // NW
    n_chunks = rows_per_w // CHUNK_R
    mesh = plsc.VectorSubcoreMesh(core_axis_name="c", subcore_axis_name="s")

    @functools.partial(
        pl.kernel,
        mesh=mesh,
        out_type=jax.ShapeDtypeStruct((NW, 48), jnp.float32),
        scratch_types=[
            pltpu.VMEM((CHUNK_R * n_cols,), jnp.float32),
            pltpu.VMEM((CHUNK_R + 16,), jnp.int32),
            pltpu.VMEM((2 * L,), jnp.float32),
            pltpu.VMEM((48,), jnp.float32),
        ],
    )
    def sc_partial(logits_hbm, labels_hbm, bounds_hbm, out_hbm,
                   buf, labbuf, bbuf, stat):
        cid = lax.axis_index("c")
        sid = lax.axis_index("s")
        wid = sid * 2 + cid
        row0 = row0_global + wid * rows_per_w

        pltpu.sync_copy(bounds_hbm, bbuf)
        lo_v = bbuf[pl.ds(0, 16)]
        hi_v = bbuf[pl.ds(16, 16)]
        lane = lax.iota(jnp.int32, 16)

        def chunk_body(ck, carry):
            cnt_v, asum_v, csum_v = carry
            rbase = row0 + ck * CHUNK_R
            pltpu.sync_copy(
                logits_hbm.at[pl.ds(rbase * n_cols, CHUNK_R * n_cols)], buf)
            pltpu.sync_copy(labels_hbm.at[pl.ds(rbase, CHUNK_R)],
                            labbuf.at[pl.ds(0, CHUNK_R)])

            def row_body(r, rcarry):
                cnt_v, asum_v, csum_v = rcarry
                rb = r * n_cols
                lab_v = labbuf[pl.ds(r, 16)]
                label = lab_v[0]

                def p1(j, mx):
                    x = buf[pl.ds(rb + j * 16, 16)]
                    return jnp.maximum(mx, x)

                mx = lax.fori_loop(1, 62, p1, buf[pl.ds(rb, 16)])
                xt = buf[pl.ds(rb + 984, 16)]
                tmask = lane >= 8
                mx = jnp.maximum(mx, jnp.where(tmask, xt, NEG))
                m_v = _butterfly(mx, jnp.maximum)

                def p2(j, c2):
                    s_acc, l_acc = c2
                    x = buf[pl.ds(rb + j * 16, 16)]
                    s_acc = s_acc + jnp.exp(x - m_v)
                    l_acc = jnp.maximum(
                        l_acc, jnp.where(j * 16 + lane == label, x, NEG))
                    return s_acc, l_acc

                x0 = buf[pl.ds(rb, 16)]
                s0 = jnp.exp(x0 - m_v)
                l0 = jnp.where(lane == label, x0, NEG)
                s_acc, l_acc = lax.fori_loop(1, 62, p2, (s0, l0))
                s_acc = s_acc + jnp.where(tmask, jnp.exp(xt - m_v), 0.0)
                l_acc = jnp.maximum(
                    l_acc,
                    jnp.where(tmask & (984 + lane == label), xt, NEG))

                s_v = _butterfly(s_acc, jnp.add)
                xl_v = _butterfly(l_acc, jnp.maximum)
                conf_v = 1.0 / s_v
                acc_v = jnp.where(xl_v == m_v, 1.0, 0.0)

                onehot = (conf_v > lo_v) & (conf_v <= hi_v)
                cnt_v = cnt_v + jnp.where(onehot, 1.0, 0.0)
                asum_v = asum_v + jnp.where(onehot, acc_v, 0.0)
                csum_v = csum_v + jnp.where(onehot, conf_v, 0.0)
                return cnt_v, asum_v, csum_v

            return lax.fori_loop(0, CHUNK_R, row_body, (cnt_v, asum_v, csum_v))

        z = jnp.zeros((L,), jnp.float32)
        cnt_v, asum_v, csum_v = lax.fori_loop(0, n_chunks, chunk_body,
                                              (z, z, z))
        stat[pl.ds(0, 16)] = cnt_v
        stat[pl.ds(16, 16)] = asum_v
        stat[pl.ds(32, 16)] = csum_v
        pltpu.sync_copy(stat, out_hbm.at[wid])

    return sc_partial


def _merge_kernel(n_rows, cnt_ref, asum_ref, csum_ref, sc_ref,
                  ece_ref, accs_ref, confs_ref):
    scp = jnp.sum(sc_ref[...], axis=0, keepdims=True)
    cnt = cnt_ref[...] + scp[:, 0:N_BINS]
    asum = asum_ref[...] + scp[:, 16:16 + N_BINS]
    csum = csum_ref[...] + scp[:, 32:32 + N_BINS]
    prop = cnt / jnp.float32(n_rows)
    safe = jnp.maximum(cnt, 1.0)
    acc_in = asum / safe
    conf_in = csum / safe
    nonempty = cnt > 0
    contrib = jnp.where(nonempty, jnp.abs(conf_in - acc_in) * prop, 0.0)
    ece_ref[...] = jnp.sum(contrib, axis=1, keepdims=True)
    accs_ref[...] = jnp.where(nonempty, acc_in, 0.0)
    confs_ref[...] = jnp.where(nonempty, conf_in, 0.0)


def kernel(logits, labels):
    n_rows, n_cols = logits.shape
    block_r = 512
    nb = SPLIT // (2 * block_r)
    labels3 = labels[:SPLIT].reshape(nb, 2 * block_r, 1)
    b = jnp.linspace(0.0, 1.0, N_BINS + 1).astype(jnp.float32)
    bounds2d = b.reshape(1, N_BINS + 1)

    cnt, asum, csum = _tc_partial(logits, labels3, bounds2d, nb, block_r)

    pad_lo = jnp.full((L - N_BINS,), 2.0, jnp.float32)
    pad_hi = jnp.full((L - N_BINS,), 3.0, jnp.float32)
    sc_bounds = jnp.concatenate([b[0:N_BINS], pad_lo,
                                 b[1:N_BINS + 1], pad_hi])
    sc_fn = _make_sc_partial(n_rows - SPLIT, SPLIT, n_cols)
    sc_out = sc_fn(logits.reshape(-1), labels, sc_bounds)

    ece2, accs2, confs2 = pl.pallas_call(
        functools.partial(_merge_kernel, n_rows),
        out_shape=[
            jax.ShapeDtypeStruct((1, 1), jnp.float32),
            jax.ShapeDtypeStruct((1, N_BINS), jnp.float32),
            jax.ShapeDtypeStruct((1, N_BINS), jnp.float32),
        ],
    )(cnt, asum, csum, sc_out)
    return (ece2.reshape(1), accs2.reshape(N_BINS), confs2.reshape(N_BINS))


# speedup vs baseline: 1.3726x; 1.3726x over previous
import functools

import jax
import jax.numpy as jnp
from jax import lax
from jax.experimental import pallas as pl
from jax.experimental.pallas import tpu as pltpu
from jax.experimental.pallas import tpu_sc as plsc

N_BINS = 11
L = 16
NW = 32
CHUNK_R = 64
NEG = -3.0e38
SPLIT = 12288


def _stats(x, labels, lo, hi):
    m = jnp.max(x, axis=1, keepdims=True)
    s = jnp.sum(jnp.exp(x - m), axis=1, keepdims=True)
    conf = 1.0 / s
    col = jax.lax.broadcasted_iota(jnp.int32, x.shape, 1)
    xl = jnp.max(jnp.where(col == labels, x, NEG), axis=1, keepdims=True)
    acc = (xl == m).astype(jnp.float32)
    mask = ((conf > lo) & (conf <= hi)).astype(jnp.float32)
    return (jnp.sum(mask, axis=0, keepdims=True),
            jnp.sum(mask * acc, axis=0, keepdims=True),
            jnp.sum(mask * conf, axis=0, keepdims=True))


def _tc_partial_kernel(nb, xa_ref, xb_ref, labels_ref, bounds_ref,
                       cnt_ref, asum_ref, csum_ref, cnt_s, asum_s, csum_s):
    i = pl.program_id(0)

    @pl.when(i == 0)
    def _init():
        cnt_s[...] = jnp.zeros_like(cnt_s)
        asum_s[...] = jnp.zeros_like(asum_s)
        csum_s[...] = jnp.zeros_like(csum_s)

    lo = bounds_ref[0:1, 0:N_BINS]
    hi = bounds_ref[0:1, 1:N_BINS + 1]
    half = xa_ref.shape[0]
    labs = labels_ref[0]

    c1, a1, s1 = _stats(xa_ref[...], labs[:half], lo, hi)
    c2, a2, s2 = _stats(xb_ref[...], labs[half:], lo, hi)
    cnt_s[...] += c1 + c2
    asum_s[...] += a1 + a2
    csum_s[...] += s1 + s2

    @pl.when(i == nb - 1)
    def _fin():
        cnt_ref[...] = cnt_s[...]
        asum_ref[...] = asum_s[...]
        csum_ref[...] = csum_s[...]


def _tc_partial(logits, labels3, bounds, nb, block_r):
    n_cols = logits.shape[1]
    return pl.pallas_call(
        functools.partial(_tc_partial_kernel, nb),
        grid=(nb,),
        in_specs=[
            pl.BlockSpec((block_r, n_cols), lambda i: (2 * i, 0)),
            pl.BlockSpec((block_r, n_cols), lambda i: (2 * i + 1, 0)),
            pl.BlockSpec((1, 2 * block_r, 1), lambda i: (i, 0, 0)),
            pl.BlockSpec((1, N_BINS + 1), lambda i: (0, 0)),
        ],
        out_specs=[
            pl.BlockSpec((1, N_BINS), lambda i: (0, 0)),
            pl.BlockSpec((1, N_BINS), lambda i: (0, 0)),
            pl.BlockSpec((1, N_BINS), lambda i: (0, 0)),
        ],
        out_shape=[
            jax.ShapeDtypeStruct((1, N_BINS), jnp.float32),
            jax.ShapeDtypeStruct((1, N_BINS), jnp.float32),
            jax.ShapeDtypeStruct((1, N_BINS), jnp.float32),
        ],
        scratch_shapes=[
            pltpu.VMEM((1, N_BINS), jnp.float32),
            pltpu.VMEM((1, N_BINS), jnp.float32),
            pltpu.VMEM((1, N_BINS), jnp.float32),
        ],
    )(logits, logits, labels3, bounds)


def _perm(v, idx):
    return lax.gather(
        v, idx[:, None],
        lax.GatherDimensionNumbers(
            offset_dims=(), collapsed_slice_dims=(0,), start_index_map=(0,)),
        slice_sizes=(1,),
        mode=lax.GatherScatterMode.PROMISE_IN_BOUNDS)


def _butterfly(v, op):
    for sh in (8, 4, 2, 1):
        idx = (jnp.arange(16, dtype=jnp.int32) + sh) % 16
        v = op(v, _perm(v, idx))
    return v


def _make_sc_partial(n_rows_sc, row0_global, n_cols):
    rows_per_w = n_rows_sc // NW
    n_chunks = rows_per_w // CHUNK_R
    mesh = plsc.VectorSubcoreMesh(core_axis_name="c", subcore_axis_name="s")

    @functools.partial(
        pl.kernel,
        mesh=mesh,
        out_type=jax.ShapeDtypeStruct((NW, 48), jnp.float32),
        scratch_types=[
            pltpu.VMEM((CHUNK_R * n_cols,), jnp.float32),
            pltpu.VMEM((CHUNK_R + 16,), jnp.int32),
            pltpu.VMEM((2 * L,), jnp.float32),
            pltpu.VMEM((48,), jnp.float32),
        ],
    )
    def sc_partial(logits_hbm, labels_hbm, bounds_hbm, out_hbm,
                   buf, labbuf, bbuf, stat):
        cid = lax.axis_index("c")
        sid = lax.axis_index("s")
        wid = sid * 2 + cid
        row0 = row0_global + wid * rows_per_w

        pltpu.sync_copy(bounds_hbm, bbuf)
        lo_v = bbuf[pl.ds(0, 16)]
        hi_v = bbuf[pl.ds(16, 16)]
        lane = lax.iota(jnp.int32, 16)

        def chunk_body(ck, carry):
            cnt_v, asum_v, csum_v = carry
            rbase = row0 + ck * CHUNK_R
            pltpu.sync_copy(
                logits_hbm.at[pl.ds(rbase * n_cols, CHUNK_R * n_cols)], buf)
            pltpu.sync_copy(labels_hbm.at[pl.ds(rbase, CHUNK_R)],
                            labbuf.at[pl.ds(0, CHUNK_R)])

            def row_body(r, rcarry):
                cnt_v, asum_v, csum_v = rcarry
                rb = r * n_cols
                lab_v = labbuf[pl.ds(r, 16)]
                label = lab_v[0]

                xs = [buf[pl.ds(rb + j * 16, 16)] for j in range(62)]
                xt = buf[pl.ds(rb + 984, 16)]
                tmask = lane >= 8
                mxa = [xs[0], xs[1], xs[2], xs[3]]
                for j in range(4, 62):
                    mxa[j % 4] = jnp.maximum(mxa[j % 4], xs[j])
                mx = jnp.maximum(jnp.maximum(mxa[0], mxa[1]),
                                 jnp.maximum(mxa[2], mxa[3]))
                mx = jnp.maximum(mx, jnp.where(tmask, xt, NEG))
                m_v = _butterfly(mx, jnp.maximum)

                sa = [jnp.exp(xs[k] - m_v) for k in range(4)]
                la = [jnp.where(k * 16 + lane == label, xs[k], NEG)
                      for k in range(4)]
                for j in range(4, 62):
                    k = j % 4
                    sa[k] = sa[k] + jnp.exp(xs[j] - m_v)
                    la[k] = jnp.maximum(
                        la[k], jnp.where(j * 16 + lane == label, xs[j], NEG))
                s_acc = (sa[0] + sa[1]) + (sa[2] + sa[3])
                l_acc = jnp.maximum(jnp.maximum(la[0], la[1]),
                                    jnp.maximum(la[2], la[3]))
                s_acc = s_acc + jnp.where(tmask, jnp.exp(xt - m_v), 0.0)
                l_acc = jnp.maximum(
                    l_acc,
                    jnp.where(tmask & (984 + lane == label), xt, NEG))

                s_v = _butterfly(s_acc, jnp.add)
                xl_v = _butterfly(l_acc, jnp.maximum)
                conf_v = 1.0 / s_v
                acc_v = jnp.where(xl_v == m_v, 1.0, 0.0)

                onehot = (conf_v > lo_v) & (conf_v <= hi_v)
                cnt_v = cnt_v + jnp.where(onehot, 1.0, 0.0)
                asum_v = asum_v + jnp.where(onehot, acc_v, 0.0)
                csum_v = csum_v + jnp.where(onehot, conf_v, 0.0)
                return cnt_v, asum_v, csum_v

            return lax.fori_loop(0, CHUNK_R, row_body, (cnt_v, asum_v, csum_v))

        z = jnp.zeros((L,), jnp.float32)
        cnt_v, asum_v, csum_v = lax.fori_loop(0, n_chunks, chunk_body,
                                              (z, z, z))
        stat[pl.ds(0, 16)] = cnt_v
        stat[pl.ds(16, 16)] = asum_v
        stat[pl.ds(32, 16)] = csum_v
        pltpu.sync_copy(stat, out_hbm.at[wid])

    return sc_partial


def _merge_kernel(n_rows, cnt_ref, asum_ref, csum_ref, sc_ref,
                  ece_ref, accs_ref, confs_ref):
    scp = jnp.sum(sc_ref[...], axis=0, keepdims=True)
    cnt = cnt_ref[...] + scp[:, 0:N_BINS]
    asum = asum_ref[...] + scp[:, 16:16 + N_BINS]
    csum = csum_ref[...] + scp[:, 32:32 + N_BINS]
    prop = cnt / jnp.float32(n_rows)
    safe = jnp.maximum(cnt, 1.0)
    acc_in = asum / safe
    conf_in = csum / safe
    nonempty = cnt > 0
    contrib = jnp.where(nonempty, jnp.abs(conf_in - acc_in) * prop, 0.0)
    ece_ref[...] = jnp.sum(contrib, axis=1, keepdims=True)
    accs_ref[...] = jnp.where(nonempty, acc_in, 0.0)
    confs_ref[...] = jnp.where(nonempty, conf_in, 0.0)


def kernel(logits, labels):
    n_rows, n_cols = logits.shape
    block_r = 512
    nb = SPLIT // (2 * block_r)
    labels3 = labels[:SPLIT].reshape(nb, 2 * block_r, 1)
    b = jnp.linspace(0.0, 1.0, N_BINS + 1).astype(jnp.float32)
    bounds2d = b.reshape(1, N_BINS + 1)

    cnt, asum, csum = _tc_partial(logits, labels3, bounds2d, nb, block_r)

    pad_lo = jnp.full((L - N_BINS,), 2.0, jnp.float32)
    pad_hi = jnp.full((L - N_BINS,), 3.0, jnp.float32)
    sc_bounds = jnp.concatenate([b[0:N_BINS], pad_lo,
                                 b[1:N_BINS + 1], pad_hi])
    sc_fn = _make_sc_partial(n_rows - SPLIT, SPLIT, n_cols)
    sc_out = sc_fn(logits.reshape(-1), labels, sc_bounds)

    ece2, accs2, confs2 = pl.pallas_call(
        functools.partial(_merge_kernel, n_rows),
        out_shape=[
            jax.ShapeDtypeStruct((1, 1), jnp.float32),
            jax.ShapeDtypeStruct((1, N_BINS), jnp.float32),
            jax.ShapeDtypeStruct((1, N_BINS), jnp.float32),
        ],
    )(cnt, asum, csum, sc_out)
    return (ece2.reshape(1), accs2.reshape(N_BINS), confs2.reshape(N_BINS))


# speedup vs baseline: 3.1192x; 2.2725x over previous
import functools

import jax
import jax.numpy as jnp
from jax.experimental import pallas as pl
from jax.experimental.pallas import tpu as pltpu

N_BINS = 11


def _stats(x, labels, lo, hi, n_cols):
    m = jnp.max(x, axis=1, keepdims=True)
    s = jnp.sum(jnp.exp(x - m), axis=1, keepdims=True)
    conf = 1.0 / s
    col = jax.lax.broadcasted_iota(jnp.int32, x.shape, 1)
    xl = jnp.max(jnp.where(col == labels, x, -3.0e38), axis=1, keepdims=True)
    acc = (xl == m).astype(jnp.float32)
    mask = ((conf > lo) & (conf <= hi)).astype(jnp.float32)
    return (jnp.sum(mask, axis=0, keepdims=True),
            jnp.sum(mask * acc, axis=0, keepdims=True),
            jnp.sum(mask * conf, axis=0, keepdims=True))


def _ece_kernel(n_rows, n_cols, nb, xa_ref, xb_ref, labels_ref, bounds_ref,
                ece_ref, accs_ref, confs_ref, cnt_s, asum_s, csum_s):
    i = pl.program_id(0)

    @pl.when(i == 0)
    def _init():
        cnt_s[...] = jnp.zeros_like(cnt_s)
        asum_s[...] = jnp.zeros_like(asum_s)
        csum_s[...] = jnp.zeros_like(csum_s)

    lo = bounds_ref[0:1, 0:N_BINS]
    hi = bounds_ref[0:1, 1:N_BINS + 1]
    half = xa_ref.shape[0]
    labs = labels_ref[0]

    c1, a1, s1 = _stats(xa_ref[...], labs[:half], lo, hi, n_cols)
    c2, a2, s2 = _stats(xb_ref[...], labs[half:], lo, hi, n_cols)
    cnt_s[...] += c1 + c2
    asum_s[...] += a1 + a2
    csum_s[...] += s1 + s2

    @pl.when(i == nb - 1)
    def _fin():
        cnt = cnt_s[...]
        prop = cnt / jnp.float32(n_rows)
        safe = jnp.maximum(cnt, 1.0)
        acc_in = asum_s[...] / safe
        conf_in = csum_s[...] / safe
        nonempty = cnt > 0
        contrib = jnp.where(nonempty, jnp.abs(conf_in - acc_in) * prop, 0.0)
        ece_ref[...] = jnp.sum(contrib, axis=1, keepdims=True)
        accs_ref[...] = jnp.where(nonempty, acc_in, 0.0)
        confs_ref[...] = jnp.where(nonempty, conf_in, 0.0)


def kernel(logits, labels):
    n_rows, n_cols = logits.shape
    block_r = 1024
    nb = n_rows // (2 * block_r)
    labels3 = labels.reshape(nb, 2 * block_r, 1)
    bounds = jnp.linspace(0.0, 1.0, N_BINS + 1).astype(jnp.float32)
    bounds = bounds.reshape(1, N_BINS + 1)

    body = functools.partial(_ece_kernel, n_rows, n_cols, nb)
    ece2, accs2, confs2 = pl.pallas_call(
        body,
        grid=(nb,),
        in_specs=[
            pl.BlockSpec((block_r, n_cols), lambda i: (2 * i, 0)),
            pl.BlockSpec((block_r, n_cols), lambda i: (2 * i + 1, 0)),
            pl.BlockSpec((1, 2 * block_r, 1), lambda i: (i, 0, 0)),
            pl.BlockSpec((1, N_BINS + 1), lambda i: (0, 0)),
        ],
        out_specs=[
            pl.BlockSpec((1, 1), lambda i: (0, 0)),
            pl.BlockSpec((1, N_BINS), lambda i: (0, 0)),
            pl.BlockSpec((1, N_BINS), lambda i: (0, 0)),
        ],
        out_shape=[
            jax.ShapeDtypeStruct((1, 1), jnp.float32),
            jax.ShapeDtypeStruct((1, N_BINS), jnp.float32),
            jax.ShapeDtypeStruct((1, N_BINS), jnp.float32),
        ],
        scratch_shapes=[
            pltpu.VMEM((1, N_BINS), jnp.float32),
            pltpu.VMEM((1, N_BINS), jnp.float32),
            pltpu.VMEM((1, N_BINS), jnp.float32),
        ],
    )(logits, logits, labels3, bounds)
    return (ece2.reshape(1), accs2.reshape(N_BINS), confs2.reshape(N_BINS))
